# baseline (speedup 1.0000x reference)
import functools

import jax
import jax.numpy as jnp
from jax import lax
from jax.experimental import pallas as pl
from jax.experimental.pallas import tpu as pltpu
from jax.experimental.pallas import tpu_sc as plsc

N = 10000
NPAD = 10240
E = 320000
G = 8
C = 128
KB = 128
NC, NS = 2, 16
NW = NC * NS
EW = -(-E // (NW * 2 * KB)) * 2 * KB
NB = EW // KB
NB0 = 80
NB1 = 2 * NB - NB0
RPS = NPAD // NS
R = 512
NR = NPAD // R


def _make_segprop(nchunks, cw, gather=True):
  mesh = plsc.VectorSubcoreMesh(core_axis_name="c", subcore_axis_name="s")
  out_type = tuple(
      jax.ShapeDtypeStruct((NC, NPAD, cw), jnp.float32) for _ in range(nchunks))
  scratch = [
      pltpu.VMEM((NB, KB), jnp.int32),
      pltpu.VMEM((NB, KB), jnp.int32),
      pltpu.VMEM((KB, cw), jnp.float32),
      pltpu.VMEM_SHARED((NPAD, cw), jnp.float32),
      pltpu.SemaphoreType.DMA,
      pltpu.SemaphoreType.DMA,
  ]

  def body(*refs):
    us = refs[:nchunks]
    srcb, dstb, zsrc = refs[nchunks], refs[nchunks + 1], refs[nchunks + 2]
    outs = refs[nchunks + 3:nchunks + 3 + nchunks]
    (src_v, dst_v, rows_v, acc, gsem, ssem) = refs[nchunks + 3 + nchunks:]
    cid = lax.axis_index("c")
    sid = lax.axis_index("s")
    wid = sid * NC + cid
    zrow = sid * RPS
    pltpu.sync_copy(srcb.at[wid], src_v)
    pltpu.sync_copy(dstb.at[wid], dst_v)

    for j in range(nchunks):
      u = us[j]
      pltpu.sync_copy(zsrc, acc.at[pl.ds(zrow, RPS)])
      if not gather:
        pltpu.sync_copy(u.at[pl.ds(0, KB)], rows_v)
      plsc.subcore_barrier()

      def ebody(i, _):
        if gather:
          pltpu.async_copy(u.at[src_v.at[i]], rows_v, gsem).wait()
        pltpu.async_copy(rows_v, acc.at[dst_v.at[i]], ssem, add=True).wait()
        return 0

      lax.fori_loop(0, NB, ebody, 0, unroll=False)
      plsc.subcore_barrier()
      pltpu.sync_copy(acc.at[pl.ds(zrow, RPS)],
                      outs[j].at[cid, pl.ds(zrow, RPS)])
      if j + 1 < nchunks:
        plsc.subcore_barrier()

  kern = pl.kernel(body, out_type=out_type, mesh=mesh, scratch_types=scratch,
                   name="segprop%dx%d" % (nchunks, cw))

  def run(u_chunks, srcb, dstb, zsrc):
    return kern(*u_chunks, srcb, dstb, zsrc)

  return run


_segprop = {k: _make_segprop(k, C) for k in (1, 2, 4)}
_seghist = _make_segprop(1, C, gather=False)


def _rowblk(n_minor):
  return pl.BlockSpec((R, n_minor), lambda r: (r, 0))


def _full(shape):
  nd = len(shape)
  return pl.BlockSpec(shape, lambda r: (0,) * nd)


def _onehot(batch_blk):
  g = lax.broadcasted_iota(jnp.int32, (R, G), 1)
  return jnp.where(batch_blk == g, 1.0, 0.0).astype(jnp.float32)


def _dotg(p, y):
  return lax.dot_general(p, y, (((0,), (0,)), ((), ())),
                         preferred_element_type=jnp.float32)


def _tc_dis(degp):
  def body(d_ref, o_ref):
    deg = d_ref[0, :, :1] + d_ref[1, :, :1]
    o_ref[...] = jnp.where(deg > 0, lax.rsqrt(jnp.maximum(deg, 1.0)), 0.0)

  return pl.pallas_call(
      body,
      grid=(NR,),
      in_specs=[pl.BlockSpec((NC, R, C), lambda r: (0, r, 0))],
      out_specs=_rowblk(1),
      out_shape=jax.ShapeDtypeStruct((NPAD, 1), jnp.float32),
  )(degp)


def _tc_pre(h, dis, w0, di, do):
  k = di // C

  def body(h_ref, d_ref, w_ref, acc_ref, *u_refs):
    hb = h_ref[...]
    acc_ref[...] = jnp.dot(hb, w_ref[...], preferred_element_type=jnp.float32)
    u = hb * d_ref[...]
    for j in range(k):
      u_refs[j][...] = u[:, j * C:(j + 1) * C]

  return pl.pallas_call(
      body,
      grid=(NR,),
      in_specs=[_rowblk(di), _rowblk(1), _full((di, do))],
      out_specs=[_rowblk(do)] + [_rowblk(C)] * k,
      out_shape=[jax.ShapeDtypeStruct((NPAD, do), jnp.float32)] +
                [jax.ShapeDtypeStruct((NPAD, C), jnp.float32)] * k,
  )(h, dis, w0)


def _tc_mid(s1, dis, acc, w1, di, do):
  k = di // C

  def body(*refs):
    s_refs = refs[:k]
    d_ref, a_ref, w_ref = refs[k], refs[k + 1], refs[k + 2]
    acc_ref = refs[k + 3]
    u_refs = refs[k + 4:]
    db = d_ref[...]
    parts = [-db * (s_refs[j][0] + s_refs[j][1]) for j in range(k)]
    tx1 = jnp.concatenate(parts, axis=1) if k > 1 else parts[0]
    acc_ref[...] = a_ref[...] + jnp.dot(tx1, w_ref[...],
                                        preferred_element_type=jnp.float32)
    for j in range(k):
      u_refs[j][...] = db * parts[j]

  return pl.pallas_call(
      body,
      grid=(NR,),
      in_specs=[pl.BlockSpec((NC, R, C), lambda r: (0, r, 0))] * k +
               [_rowblk(1), _rowblk(do), _full((di, do))],
      out_specs=[_rowblk(do)] + [_rowblk(C)] * k,
      out_shape=[jax.ShapeDtypeStruct((NPAD, do), jnp.float32)] +
                [jax.ShapeDtypeStruct((NPAD, C), jnp.float32)] * k,
  )(*s1, dis, acc, w1)


def _tc_post(s2, dis, h, acc, w2, b, gw, gb, gms, batch, di, do):
  k = di // C

  def body(*refs):
    s_refs = refs[:k]
    (d_ref, h_ref, a_ref, w_ref, b_ref, gw_ref, gb_ref, gms_ref,
     bat_ref) = refs[k:k + 9]
    o_ref = refs[k + 9]
    y_s, c_s, msum, vsum, cnt = refs[k + 10:]
    p = pl.program_id(0)
    r = pl.program_id(1)
    pm = _onehot(bat_ref[...])

    @pl.when(p == 0)
    def _():
      db = d_ref[...]
      parts = [-2.0 * db * (s_refs[j][0] + s_refs[j][1]) for j in range(k)]
      tx2 = (jnp.concatenate(parts, axis=1) if k > 1 else parts[0]) - h_ref[...]
      y = a_ref[...] + jnp.dot(tx2, w_ref[...],
                               preferred_element_type=jnp.float32) + b_ref[...]
      y_s[pl.ds(r * R, R)] = y
      ms = _dotg(pm, y)
      cs = jnp.broadcast_to(jnp.sum(pm, axis=0)[:, None], (G, do))

      @pl.when(r == 0)
      def _():
        msum[...] = ms
        cnt[...] = cs

      @pl.when(r > 0)
      def _():
        msum[...] += ms
        cnt[...] += cs

    @pl.when(p == 1)
    def _():
      cn = jnp.maximum(cnt[...], 1.0)
      mean = msum[...] / cn
      o = y_s[pl.ds(r * R, R)] - jnp.dot(pm, mean) * gms_ref[...]
      c_s[pl.ds(r * R, R)] = o
      vs = _dotg(pm, o * o)

      @pl.when(r == 0)
      def _():
        vsum[...] = vs

      @pl.when(r > 0)
      def _():
        vsum[...] += vs

    @pl.when(p == 2)
    def _():
      cn = jnp.maximum(cnt[...], 1.0)
      std = jnp.sqrt(vsum[...] / cn + 1e-5)
      o = c_s[pl.ds(r * R, R)]
      t = gw_ref[...] * o / jnp.dot(pm, std) + gb_ref[...]
      o_ref[...] = jnp.where(t > 0, t, 0.2 * t)

  def p0map(p, r):
    return (0, jnp.where(p == 0, r, 0), 0)

  def p0row(p, r):
    return (jnp.where(p == 0, r, 0), 0)

  return pl.pallas_call(
      body,
      grid=(3, NR),
      in_specs=[pl.BlockSpec((NC, R, C), p0map)] * k + [
          pl.BlockSpec((R, 1), p0row),
          pl.BlockSpec((R, di), p0row),
          pl.BlockSpec((R, do), p0row),
          pl.BlockSpec((di, do), lambda p, r: (0, 0)),
          pl.BlockSpec((1, do), lambda p, r: (0, 0)),
          pl.BlockSpec((1, do), lambda p, r: (0, 0)),
          pl.BlockSpec((1, do), lambda p, r: (0, 0)),
          pl.BlockSpec((1, do), lambda p, r: (0, 0)),
          pl.BlockSpec((R, 1), lambda p, r: (r, 0)),
      ],
      out_specs=pl.BlockSpec((R, do), lambda p, r: (jnp.where(p == 2, r, 0), 0)),
      out_shape=jax.ShapeDtypeStruct((NPAD, do), jnp.float32),
      scratch_shapes=[
          pltpu.VMEM((NPAD, do), jnp.float32),
          pltpu.VMEM((NPAD, do), jnp.float32),
          pltpu.VMEM((G, do), jnp.float32),
          pltpu.VMEM((G, do), jnp.float32),
          pltpu.VMEM((G, do), jnp.float32),
      ],
  )(*s2, dis, h, acc, w2, b, gw, gb, gms, batch)


def _tc_head(h4, x, batch, l1w, l1b, l2w, l2b):
  def body(h_ref, x_ref, bat_ref, w1_ref, b1_ref, w2_ref, b2_ref, o_ref,
           psum, cnt):
    r = pl.program_id(0)
    pm = _onehot(bat_ref[...])
    ps = _dotg(pm, h_ref[...] + x_ref[...])
    cs = jnp.broadcast_to(jnp.sum(pm, axis=0)[:, None], (G, 128))

    @pl.when(r == 0)
    def _():
      psum[...] = ps
      cnt[...] = cs

    @pl.when(r > 0)
    def _():
      psum[...] += ps
      cnt[...] += cs

    @pl.when(r == NR - 1)
    def _():
      pooled = psum[...] / jnp.maximum(cnt[...], 1.0)
      a = jnp.tanh(jnp.dot(pooled, w1_ref[...],
                           preferred_element_type=jnp.float32) + b1_ref[...])
      o_ref[...] = jnp.dot(a, w2_ref[...],
                           preferred_element_type=jnp.float32) + b2_ref[...]

  return pl.pallas_call(
      body,
      grid=(NR,),
      in_specs=[_rowblk(128), _rowblk(128), _rowblk(1), _full((128, 64)),
                _full((1, 64)), _full((64, 10)), _full((1, 10))],
      out_specs=pl.BlockSpec((G, 10), lambda r: (0, 0)),
      out_shape=jax.ShapeDtypeStruct((G, 10), jnp.float32),
      scratch_shapes=[pltpu.VMEM((G, 128), jnp.float32),
                      pltpu.VMEM((G, 128), jnp.float32)],
  )(h4, x, batch, l1w, l1b, l2w, l2b)


def _layer(h, dis, srcb, dstb, zsrc, w, b, gw, gb, gms, batch, di, do):
  k = di // C
  pre = _tc_pre(h, dis, w[0], di, do)
  acc, u0 = pre[0], pre[1:]
  s1 = _segprop[k](u0, srcb, dstb, zsrc)
  mid = _tc_mid(s1, dis, acc, w[1], di, do)
  acc, u1 = mid[0], mid[1:]
  s2 = _segprop[k](u1, srcb, dstb, zsrc)
  return _tc_post(s2, dis, h, acc, w[2], b, gw, gb, gms, batch, di, do)


@jax.jit
def kernel(x, edge_index, batch,
           conv1_w, conv1_b, gn1_w, gn1_b, gn1_ms,
           conv2_w, conv2_b, gn2_w, gn2_b, gn2_ms,
           conv3_w, conv3_b, gn3_w, gn3_b, gn3_ms,
           conv4_w, conv4_b, gn4_w, gn4_b, gn4_ms,
           lin1_w, lin1_b, lin2_w, lin2_b):
  src, dst = edge_index[0], edge_index[1]
  padn = NW * EW - E
  pad = jnp.full((padn,), N, jnp.int32)

  def blocked(ix):
    return jnp.concatenate([ix, pad]).reshape(NW, NB, KB)

  srcb = blocked(src)
  dstb = blocked(dst)
  srcb_as_dst = blocked(src)
  zsrc = jnp.zeros((RPS, C), jnp.float32)
  ones_tab = jnp.ones((NPAD, C), jnp.float32)

  xp = jnp.zeros((NPAD, 128), jnp.float32).at[:N].set(x)
  batp = jnp.full((NPAD, 1), G, jnp.int32).at[:N, 0].set(batch)

  degp = _segprop[1]((ones_tab,), srcb, srcb_as_dst, zsrc)[0]
  dis = _tc_dis(degp)

  h = xp
  dims = [(128, 256), (256, 512), (512, 256), (256, 128)]
  params = [(conv1_w, conv1_b, gn1_w, gn1_b, gn1_ms),
            (conv2_w, conv2_b, gn2_w, gn2_b, gn2_ms),
            (conv3_w, conv3_b, gn3_w, gn3_b, gn3_ms),
            (conv4_w, conv4_b, gn4_w, gn4_b, gn4_ms)]
  for (di, do), (w, b, gw, gb, gms) in zip(dims, params):
    h = _layer(h, dis, srcb, dstb, zsrc, w, b.reshape(1, do),
               gw.reshape(1, do), gb.reshape(1, do), gms.reshape(1, do),
               batp, di, do)

  return _tc_head(h, xp, batp, lin1_w, lin1_b.reshape(1, 64),
                  lin2_w, lin2_b.reshape(1, 10))


# speedup vs baseline: 1.6170x; 1.6170x over previous
import functools

import jax
import jax.numpy as jnp
from jax import lax
from jax.experimental import pallas as pl
from jax.experimental.pallas import tpu as pltpu
from jax.experimental.pallas import tpu_sc as plsc

N = 10000
NPAD = 10240
E = 320000
G = 8
C = 128
KB = 128
NC, NS = 2, 16
NW = NC * NS
EW = -(-E // (NW * KB)) * KB
NB = EW // KB
RPS = NPAD // NS
R = 512
NR = NPAD // R


def _make_segprop(nchunks, cw, gather=True):
  mesh = plsc.VectorSubcoreMesh(core_axis_name="c", subcore_axis_name="s")
  out_type = tuple(
      jax.ShapeDtypeStruct((NC, NPAD, cw), jnp.float32) for _ in range(nchunks))
  scratch = [
      pltpu.VMEM((NB, KB), jnp.int32),
      pltpu.VMEM((NB, KB), jnp.int32),
      pltpu.VMEM((KB, cw), jnp.float32),
      pltpu.VMEM_SHARED((NPAD, cw), jnp.float32),
      pltpu.SemaphoreType.DMA,
      pltpu.SemaphoreType.DMA,
  ]

  def body(*refs):
    us = refs[:nchunks]
    srcb, dstb, zsrc = refs[nchunks], refs[nchunks + 1], refs[nchunks + 2]
    outs = refs[nchunks + 3:nchunks + 3 + nchunks]
    (src_v, dst_v, rows_v, acc, gsem, ssem) = refs[nchunks + 3 + nchunks:]
    cid = lax.axis_index("c")
    sid = lax.axis_index("s")
    wid = sid * NC + cid
    zrow = sid * RPS
    pltpu.sync_copy(srcb.at[wid], src_v)
    pltpu.sync_copy(dstb.at[wid], dst_v)

    for j in range(nchunks):
      u = us[j]
      pltpu.sync_copy(zsrc, acc.at[pl.ds(zrow, RPS)])
      if not gather:
        pltpu.sync_copy(u.at[pl.ds(0, KB)], rows_v)
      plsc.subcore_barrier()

      def ebody(i, _):
        if gather:
          pltpu.async_copy(u.at[src_v.at[i]], rows_v, gsem).wait()
        pltpu.async_copy(rows_v, acc.at[dst_v.at[i]], ssem, add=True).wait()
        return 0

      lax.fori_loop(0, NB, ebody, 0, unroll=False)
      plsc.subcore_barrier()
      pltpu.sync_copy(acc.at[pl.ds(zrow, RPS)],
                      outs[j].at[cid, pl.ds(zrow, RPS)])
      if j + 1 < nchunks:
        plsc.subcore_barrier()

  kern = pl.kernel(body, out_type=out_type, mesh=mesh, scratch_types=scratch,
                   name="segprop%dx%d" % (nchunks, cw))

  def run(u_chunks, srcb, dstb, zsrc):
    return kern(*u_chunks, srcb, dstb, zsrc)

  return run


_segprop = {k: _make_segprop(k, C) for k in (1, 2, 4)}
_seghist = _make_segprop(1, C, gather=False)


def _rowblk(n_minor):
  return pl.BlockSpec((R, n_minor), lambda r: (r, 0))


def _full(shape):
  nd = len(shape)
  return pl.BlockSpec(shape, lambda r: (0,) * nd)


def _onehot(batch_blk):
  g = lax.broadcasted_iota(jnp.int32, (R, G), 1)
  return jnp.where(batch_blk == g, 1.0, 0.0).astype(jnp.float32)


def _dotg(p, y):
  return lax.dot_general(p, y, (((0,), (0,)), ((), ())),
                         preferred_element_type=jnp.float32)


def _tc_dis(degp):
  def body(d_ref, o_ref):
    deg = d_ref[0, :, :1] + d_ref[1, :, :1]
    o_ref[...] = jnp.where(deg > 0, lax.rsqrt(jnp.maximum(deg, 1.0)), 0.0)

  return pl.pallas_call(
      body,
      grid=(NR,),
      in_specs=[pl.BlockSpec((NC, R, C), lambda r: (0, r, 0))],
      out_specs=_rowblk(1),
      out_shape=jax.ShapeDtypeStruct((NPAD, 1), jnp.float32),
  )(degp)


def _tc_pre(h, dis, w0, di, do):
  k = di // C

  def body(h_ref, d_ref, w_ref, acc_ref, *u_refs):
    hb = h_ref[...]
    acc_ref[...] = jnp.dot(hb, w_ref[...], preferred_element_type=jnp.float32)
    u = hb * d_ref[...]
    for j in range(k):
      u_refs[j][...] = u[:, j * C:(j + 1) * C]

  return pl.pallas_call(
      body,
      grid=(NR,),
      in_specs=[_rowblk(di), _rowblk(1), _full((di, do))],
      out_specs=[_rowblk(do)] + [_rowblk(C)] * k,
      out_shape=[jax.ShapeDtypeStruct((NPAD, do), jnp.float32)] +
                [jax.ShapeDtypeStruct((NPAD, C), jnp.float32)] * k,
  )(h, dis, w0)


def _tc_mid(s1, dis, acc, w1, di, do):
  k = di // C

  def body(*refs):
    s_refs = refs[:k]
    d_ref, a_ref, w_ref = refs[k], refs[k + 1], refs[k + 2]
    acc_ref = refs[k + 3]
    u_refs = refs[k + 4:]
    db = d_ref[...]
    parts = [-db * (s_refs[j][0] + s_refs[j][1]) for j in range(k)]
    tx1 = jnp.concatenate(parts, axis=1) if k > 1 else parts[0]
    acc_ref[...] = a_ref[...] + jnp.dot(tx1, w_ref[...],
                                        preferred_element_type=jnp.float32)
    for j in range(k):
      u_refs[j][...] = db * parts[j]

  return pl.pallas_call(
      body,
      grid=(NR,),
      in_specs=[pl.BlockSpec((NC, R, C), lambda r: (0, r, 0))] * k +
               [_rowblk(1), _rowblk(do), _full((di, do))],
      out_specs=[_rowblk(do)] + [_rowblk(C)] * k,
      out_shape=[jax.ShapeDtypeStruct((NPAD, do), jnp.float32)] +
                [jax.ShapeDtypeStruct((NPAD, C), jnp.float32)] * k,
  )(*s1, dis, acc, w1)


def _tc_post(s2, dis, h, acc, w2, b, gw, gb, gms, batch, di, do):
  k = di // C

  def body(*refs):
    s_refs = refs[:k]
    (d_ref, h_ref, a_ref, w_ref, b_ref, gw_ref, gb_ref, gms_ref,
     bat_ref) = refs[k:k + 9]
    o_ref = refs[k + 9]
    y_s, c_s, msum, vsum, cnt = refs[k + 10:]
    p = pl.program_id(0)
    r = pl.program_id(1)
    pm = _onehot(bat_ref[...])

    @pl.when(p == 0)
    def _():
      db = d_ref[...]
      parts = [-2.0 * db * (s_refs[j][0] + s_refs[j][1]) for j in range(k)]
      tx2 = (jnp.concatenate(parts, axis=1) if k > 1 else parts[0]) - h_ref[...]
      y = a_ref[...] + jnp.dot(tx2, w_ref[...],
                               preferred_element_type=jnp.float32) + b_ref[...]
      y_s[pl.ds(r * R, R)] = y
      ms = _dotg(pm, y)
      cs = jnp.broadcast_to(jnp.sum(pm, axis=0)[:, None], (G, do))

      @pl.when(r == 0)
      def _():
        msum[...] = ms
        cnt[...] = cs

      @pl.when(r > 0)
      def _():
        msum[...] += ms
        cnt[...] += cs

    @pl.when(p == 1)
    def _():
      cn = jnp.maximum(cnt[...], 1.0)
      mean = msum[...] / cn
      o = y_s[pl.ds(r * R, R)] - jnp.dot(pm, mean) * gms_ref[...]
      c_s[pl.ds(r * R, R)] = o
      vs = _dotg(pm, o * o)

      @pl.when(r == 0)
      def _():
        vsum[...] = vs

      @pl.when(r > 0)
      def _():
        vsum[...] += vs

    @pl.when(p == 2)
    def _():
      cn = jnp.maximum(cnt[...], 1.0)
      std = jnp.sqrt(vsum[...] / cn + 1e-5)
      o = c_s[pl.ds(r * R, R)]
      t = gw_ref[...] * o / jnp.dot(pm, std) + gb_ref[...]
      o_ref[...] = jnp.where(t > 0, t, 0.2 * t)

  def p0map(p, r):
    return (0, jnp.where(p == 0, r, 0), 0)

  def p0row(p, r):
    return (jnp.where(p == 0, r, 0), 0)

  return pl.pallas_call(
      body,
      grid=(3, NR),
      in_specs=[pl.BlockSpec((NC, R, C), p0map)] * k + [
          pl.BlockSpec((R, 1), p0row),
          pl.BlockSpec((R, di), p0row),
          pl.BlockSpec((R, do), p0row),
          pl.BlockSpec((di, do), lambda p, r: (0, 0)),
          pl.BlockSpec((1, do), lambda p, r: (0, 0)),
          pl.BlockSpec((1, do), lambda p, r: (0, 0)),
          pl.BlockSpec((1, do), lambda p, r: (0, 0)),
          pl.BlockSpec((1, do), lambda p, r: (0, 0)),
          pl.BlockSpec((R, 1), lambda p, r: (r, 0)),
      ],
      out_specs=pl.BlockSpec((R, do), lambda p, r: (jnp.where(p == 2, r, 0), 0)),
      out_shape=jax.ShapeDtypeStruct((NPAD, do), jnp.float32),
      scratch_shapes=[
          pltpu.VMEM((NPAD, do), jnp.float32),
          pltpu.VMEM((NPAD, do), jnp.float32),
          pltpu.VMEM((G, do), jnp.float32),
          pltpu.VMEM((G, do), jnp.float32),
          pltpu.VMEM((G, do), jnp.float32),
      ],
  )(*s2, dis, h, acc, w2, b, gw, gb, gms, batch)


def _tc_head(h4, x, batch, l1w, l1b, l2w, l2b):
  def body(h_ref, x_ref, bat_ref, w1_ref, b1_ref, w2_ref, b2_ref, o_ref,
           psum, cnt):
    r = pl.program_id(0)
    pm = _onehot(bat_ref[...])
    ps = _dotg(pm, h_ref[...] + x_ref[...])
    cs = jnp.broadcast_to(jnp.sum(pm, axis=0)[:, None], (G, 128))

    @pl.when(r == 0)
    def _():
      psum[...] = ps
      cnt[...] = cs

    @pl.when(r > 0)
    def _():
      psum[...] += ps
      cnt[...] += cs

    @pl.when(r == NR - 1)
    def _():
      pooled = psum[...] / jnp.maximum(cnt[...], 1.0)
      a = jnp.tanh(jnp.dot(pooled, w1_ref[...],
                           preferred_element_type=jnp.float32) + b1_ref[...])
      o_ref[...] = jnp.dot(a, w2_ref[...],
                           preferred_element_type=jnp.float32) + b2_ref[...]

  return pl.pallas_call(
      body,
      grid=(NR,),
      in_specs=[_rowblk(128), _rowblk(128), _rowblk(1), _full((128, 64)),
                _full((1, 64)), _full((64, 10)), _full((1, 10))],
      out_specs=pl.BlockSpec((G, 10), lambda r: (0, 0)),
      out_shape=jax.ShapeDtypeStruct((G, 10), jnp.float32),
      scratch_shapes=[pltpu.VMEM((G, 128), jnp.float32),
                      pltpu.VMEM((G, 128), jnp.float32)],
  )(h4, x, batch, l1w, l1b, l2w, l2b)


def _layer(h, dis, srcb, dstb, zsrc, w, b, gw, gb, gms, batch, di, do):
  k = di // C
  pre = _tc_pre(h, dis, w[0], di, do)
  acc, u0 = pre[0], pre[1:]
  s1 = _segprop[k](u0, srcb, dstb, zsrc)
  mid = _tc_mid(s1, dis, acc, w[1], di, do)
  acc, u1 = mid[0], mid[1:]
  s2 = _segprop[k](u1, srcb, dstb, zsrc)
  return _tc_post(s2, dis, h, acc, w[2], b, gw, gb, gms, batch, di, do)


@jax.jit
def kernel(x, edge_index, batch,
           conv1_w, conv1_b, gn1_w, gn1_b, gn1_ms,
           conv2_w, conv2_b, gn2_w, gn2_b, gn2_ms,
           conv3_w, conv3_b, gn3_w, gn3_b, gn3_ms,
           conv4_w, conv4_b, gn4_w, gn4_b, gn4_ms,
           lin1_w, lin1_b, lin2_w, lin2_b):
  src, dst = edge_index[0], edge_index[1]
  padn = NW * EW - E
  pad = jnp.full((padn,), N, jnp.int32)

  def blocked(ix):
    return jnp.concatenate([ix, pad]).reshape(NW, NB, KB)

  srcb = blocked(src)
  dstb = blocked(dst)
  srcb_as_dst = blocked(src)
  zsrc = jnp.zeros((RPS, C), jnp.float32)
  ones_tab = jnp.ones((NPAD, C), jnp.float32)

  xp = jnp.zeros((NPAD, 128), jnp.float32).at[:N].set(x)
  batp = jnp.full((NPAD, 1), G, jnp.int32).at[:N, 0].set(batch)

  degp = _seghist((ones_tab,), srcb, srcb_as_dst, zsrc)[0]
  dis = _tc_dis(degp)

  h = xp
  dims = [(128, 256), (256, 512), (512, 256), (256, 128)]
  params = [(conv1_w, conv1_b, gn1_w, gn1_b, gn1_ms),
            (conv2_w, conv2_b, gn2_w, gn2_b, gn2_ms),
            (conv3_w, conv3_b, gn3_w, gn3_b, gn3_ms),
            (conv4_w, conv4_b, gn4_w, gn4_b, gn4_ms)]
  for (di, do), (w, b, gw, gb, gms) in zip(dims, params):
    h = _layer(h, dis, srcb, dstb, zsrc, w, b.reshape(1, do),
               gw.reshape(1, do), gb.reshape(1, do), gms.reshape(1, do),
               batp, di, do)

  return _tc_head(h, xp, batp, lin1_w, lin1_b.reshape(1, 64),
                  lin2_w, lin2_b.reshape(1, 10))
